# baseline (speedup 1.0000x reference)
import functools

import jax
import jax.numpy as jnp
from jax import lax
from jax.experimental import pallas as pl
from jax.experimental.pallas import tpu as pltpu
from jax.experimental.pallas import tpu_sc as plsc

_BSZ, _SEQ, _D = 8, 2048, 1024
_WSEG = 16
_NW = _SEQ // _WSEG
_NWT = _NW // 16
_AW = _WSEG * _D
_SUP = 32


def _sc_body(meta_hbm, pack_hbm, hid_hbm, zer_hbm, out_hbm,
             acc, inbuf, zbuf, sidx_v, meta_v, sw0, sw1):
    c = lax.axis_index("c")
    s = lax.axis_index("s")

    qiota = lax.iota(jnp.int32, 16)

    def _extract(vec, j):
        return jnp.max(jnp.where(qiota == j, vec, jnp.int32(-2**31)))

    pltpu.sync_copy(zer_hbm, zbuf)

    def _window(g, carry):
        b = c * 4 + g // _NWT
        wl = g % _NWT
        w = s + wl * 16
        p = g & 1

        @pl.when(wl == 0)
        def _():
            mo = pl.multiple_of((b * 16 + s) * 128, 128)
            pltpu.sync_copy(meta_hbm.at[pl.ds(mo, 128)], meta_v)

        mrow = meta_v[pl.ds(wl * 16, 16)]
        t0 = _extract(mrow, 0)
        nsc = _extract(mrow, 1)

        o0 = pl.multiple_of((b * _SEQ + w * _WSEG) * _D, _AW)

        @pl.when((g >= 2) & (p == 0))
        def _():
            pltpu.make_async_copy(
                zer_hbm, acc.at[pl.ds(0, _AW)], sw0).wait()

        @pl.when((g >= 2) & (p == 1))
        def _():
            pltpu.make_async_copy(
                zer_hbm, acc.at[pl.ds(_AW, _AW)], sw1).wait()

        @pl.when((nsc == 0) & (p == 0))
        def _():
            pltpu.make_async_copy(
                zbuf, out_hbm.at[pl.ds(o0, _AW)], sw0).start()

        @pl.when((nsc == 0) & (p == 1))
        def _():
            pltpu.make_async_copy(
                zbuf, out_hbm.at[pl.ds(o0, _AW)], sw1).start()

        @pl.when(nsc > 0)
        def _():
            pltpu.sync_copy(zer_hbm, acc.at[pl.ds(p * _AW, _AW)])

            def _batch(k, carry2):
                cb = pl.multiple_of(
                    (b * _SEQ + t0 + k * _SUP) * _D, _SUP * _D)
                pltpu.sync_copy(hid_hbm.at[pl.ds(cb, _SUP * _D)], inbuf)
                cp = pl.multiple_of(b * _SEQ + t0 + k * _SUP, _SUP)
                pltpu.sync_copy(pack_hbm.at[pl.ds(cp, _SUP)], sidx_v)
                for h in range(2):
                    sv = sidx_v[pl.ds(h * 16, 16)]
                    for j in range(16):
                        sb = _extract(sv, j)

                        @pl.when((sb >> 16) == w)
                        def _():
                            base = p * _AW + (sb & 0xFFFF) * _D

                            def _q2(q, carry3):
                                for u in range(32):
                                    off = q * 512 + u * 16
                                    a = pl.ds(base + off, 16)
                                    acc[a] = acc[a] + inbuf[pl.ds(
                                        (h * 16 + j) * _D + off, 16)]
                                return carry3
                            lax.fori_loop(0, 2, _q2, 0)
                return carry2
            lax.fori_loop(0, nsc, _batch, 0)

            @pl.when(p == 0)
            def _():
                pltpu.make_async_copy(
                    acc.at[pl.ds(0, _AW)],
                    out_hbm.at[pl.ds(o0, _AW)], sw0).start()

            @pl.when(p == 1)
            def _():
                pltpu.make_async_copy(
                    acc.at[pl.ds(_AW, _AW)],
                    out_hbm.at[pl.ds(o0, _AW)], sw1).start()
        return carry
    lax.fori_loop(0, 4 * _NWT, _window, 0)

    pltpu.make_async_copy(zer_hbm, acc.at[pl.ds(0, _AW)], sw0).wait()
    pltpu.make_async_copy(zer_hbm, acc.at[pl.ds(_AW, _AW)], sw1).wait()


def kernel(domains, hidden_layers, binary_preditions):
    del domains
    labels = binary_preditions
    is_B = (labels == 1).astype(jnp.int32)
    is_I = labels == 2
    cs = jnp.cumsum(is_B, axis=1)
    seg = cs - 1
    valid = ((is_B == 1) | is_I) & (seg >= 0)

    pack = jnp.where(valid, (seg // _WSEG) * 65536 + seg % _WSEG, -1)
    pack = pack.astype(jnp.int32).reshape(-1)

    bounds = jnp.arange(_NW + 1, dtype=jnp.int32) * _WSEG
    below = (seg[:, None, :] < bounds[None, :, None]).sum(-1)
    t_lo = (below[:, :-1] // _SUP) * _SUP
    nsc = (below[:, 1:] - t_lo + _SUP - 1) // _SUP
    fields = jnp.stack(
        [t_lo, nsc] + [jnp.zeros_like(t_lo)] * 14, axis=-1)
    meta = (fields.reshape(_BSZ, _NWT, 16, 16)
            .transpose(0, 2, 1, 3)
            .reshape(-1).astype(jnp.int32))

    hid_flat = hidden_layers.reshape(-1)
    zer = jnp.zeros((_AW,), jnp.float32)

    mesh = plsc.VectorSubcoreMesh(core_axis_name="c", subcore_axis_name="s")
    sc = functools.partial(
        pl.kernel,
        mesh=mesh,
        compiler_params=pltpu.CompilerParams(needs_layout_passes=False),
        out_type=jax.ShapeDtypeStruct((_BSZ * _SEQ * _D,), jnp.float32),
        scratch_types=[
            pltpu.VMEM((2 * _AW,), jnp.float32),
            pltpu.VMEM((_SUP * _D,), jnp.float32),
            pltpu.VMEM((_AW,), jnp.float32),
            pltpu.VMEM((_SUP,), jnp.int32),
            pltpu.VMEM((128,), jnp.int32),
            pltpu.SemaphoreType.DMA,
            pltpu.SemaphoreType.DMA,
        ],
    )(_sc_body)
    return sc(meta, pack, hid_flat, zer).reshape(_BSZ * _SEQ, _D)


# speedup vs baseline: 1.0733x; 1.0733x over previous
import functools

import jax
import jax.numpy as jnp
from jax import lax
from jax.experimental import pallas as pl
from jax.experimental.pallas import tpu as pltpu
from jax.experimental.pallas import tpu_sc as plsc

_BSZ, _SEQ, _D = 8, 2048, 1024
_WSEG = 16
_NW = _SEQ // _WSEG
_NWT = _NW // 16
_AW = _WSEG * _D
_SUP = 32


def _sc_body(meta_hbm, pack_hbm, hid_hbm, zer_hbm, out_hbm,
             acc, inbuf, zbuf, sidx_v, meta_v, sw0, sw1):
    c = lax.axis_index("c")
    s = lax.axis_index("s")

    qiota = lax.iota(jnp.int32, 16)

    def _extract(vec, j):
        return jnp.max(jnp.where(qiota == j, vec, jnp.int32(-2**31)))

    pltpu.sync_copy(zer_hbm, zbuf)

    def _window(g, carry):
        b = c * 4 + g // _NWT
        wl = g % _NWT
        w = s + wl * 16
        p = g & 1

        @pl.when(wl == 0)
        def _():
            mo = pl.multiple_of((b * 16 + s) * 128, 128)
            pltpu.sync_copy(meta_hbm.at[pl.ds(mo, 128)], meta_v)

        mrow = meta_v[pl.ds(wl * 16, 16)]
        t0 = _extract(mrow, 0)
        nsc = _extract(mrow, 1)

        o0 = pl.multiple_of((b * _SEQ + w * _WSEG) * _D, _AW)

        @pl.when((g >= 2) & (p == 0))
        def _():
            pltpu.make_async_copy(
                zer_hbm, acc.at[pl.ds(0, _AW)], sw0).wait()

        @pl.when((g >= 2) & (p == 1))
        def _():
            pltpu.make_async_copy(
                zer_hbm, acc.at[pl.ds(_AW, _AW)], sw1).wait()

        @pl.when((nsc == 0) & (p == 0))
        def _():
            pltpu.make_async_copy(
                zbuf, out_hbm.at[pl.ds(o0, _AW)], sw0).start()

        @pl.when((nsc == 0) & (p == 1))
        def _():
            pltpu.make_async_copy(
                zbuf, out_hbm.at[pl.ds(o0, _AW)], sw1).start()

        @pl.when(nsc > 0)
        def _():
            pltpu.sync_copy(zer_hbm, acc.at[pl.ds(p * _AW, _AW)])

            def _batch(k, carry2):
                cb = pl.multiple_of(
                    (b * _SEQ + t0 + k * _SUP) * _D, _SUP * _D)
                pltpu.sync_copy(hid_hbm.at[pl.ds(cb, _SUP * _D)], inbuf)
                cp = pl.multiple_of(b * _SEQ + t0 + k * _SUP, _SUP)
                pltpu.sync_copy(pack_hbm.at[pl.ds(cp, _SUP)], sidx_v)
                for h in range(2):
                    sv = sidx_v[pl.ds(h * 16, 16)]
                    for j in range(16):
                        sb = _extract(sv, j)

                        @pl.when((sb >> 16) == w)
                        def _():
                            base = jnp.full(
                                (16,),
                                p * _AW + (sb & 0xFFFF) * _D,
                                jnp.int32) + qiota
                            for u in range(64):
                                plsc.addupdate_scatter(
                                    acc, [base + u * 16],
                                    inbuf[pl.ds(
                                        (h * 16 + j) * _D + u * 16, 16)])
                return carry2
            lax.fori_loop(0, nsc, _batch, 0)

            @pl.when(p == 0)
            def _():
                pltpu.make_async_copy(
                    acc.at[pl.ds(0, _AW)],
                    out_hbm.at[pl.ds(o0, _AW)], sw0).start()

            @pl.when(p == 1)
            def _():
                pltpu.make_async_copy(
                    acc.at[pl.ds(_AW, _AW)],
                    out_hbm.at[pl.ds(o0, _AW)], sw1).start()
        return carry
    lax.fori_loop(0, 4 * _NWT, _window, 0)

    pltpu.make_async_copy(zer_hbm, acc.at[pl.ds(0, _AW)], sw0).wait()
    pltpu.make_async_copy(zer_hbm, acc.at[pl.ds(_AW, _AW)], sw1).wait()


def kernel(domains, hidden_layers, binary_preditions):
    del domains
    labels = binary_preditions
    is_B = (labels == 1).astype(jnp.int32)
    is_I = labels == 2
    cs = jnp.cumsum(is_B, axis=1)
    seg = cs - 1
    valid = ((is_B == 1) | is_I) & (seg >= 0)

    pack = jnp.where(valid, (seg // _WSEG) * 65536 + seg % _WSEG, -1)
    pack = pack.astype(jnp.int32).reshape(-1)

    bounds = jnp.arange(_NW + 1, dtype=jnp.int32) * _WSEG
    below = (seg[:, None, :] < bounds[None, :, None]).sum(-1)
    t_lo = (below[:, :-1] // _SUP) * _SUP
    nsc = (below[:, 1:] - t_lo + _SUP - 1) // _SUP
    fields = jnp.stack(
        [t_lo, nsc] + [jnp.zeros_like(t_lo)] * 14, axis=-1)
    meta = (fields.reshape(_BSZ, _NWT, 16, 16)
            .transpose(0, 2, 1, 3)
            .reshape(-1).astype(jnp.int32))

    hid_flat = hidden_layers.reshape(-1)
    zer = jnp.zeros((_AW,), jnp.float32)

    mesh = plsc.VectorSubcoreMesh(core_axis_name="c", subcore_axis_name="s")
    sc = functools.partial(
        pl.kernel,
        mesh=mesh,
        compiler_params=pltpu.CompilerParams(needs_layout_passes=False),
        out_type=jax.ShapeDtypeStruct((_BSZ * _SEQ * _D,), jnp.float32),
        scratch_types=[
            pltpu.VMEM((2 * _AW,), jnp.float32),
            pltpu.VMEM((_SUP * _D,), jnp.float32),
            pltpu.VMEM((_AW,), jnp.float32),
            pltpu.VMEM((_SUP,), jnp.int32),
            pltpu.VMEM((128,), jnp.int32),
            pltpu.SemaphoreType.DMA,
            pltpu.SemaphoreType.DMA,
        ],
    )(_sc_body)
    return sc(meta, pack, hid_flat, zer).reshape(_BSZ * _SEQ, _D)


# speedup vs baseline: 2.3266x; 2.1677x over previous
import functools

import jax
import jax.numpy as jnp
from jax import lax
from jax.experimental import pallas as pl
from jax.experimental.pallas import tpu as pltpu
from jax.experimental.pallas import tpu_sc as plsc

_BSZ, _SEQ, _D = 8, 2048, 1024
_BTC = 6
_BSC = _BSZ - _BTC

_T = 256
_SBLK = _T + 16
_NTB = _SEQ // _T

_WSEG = 16
_NW = _SEQ // _WSEG
_NWT = _NW // 16
_AW = _WSEG * _D
_SUP = 32


def _tc_kernel(base_ref, vseg_ref, h_ref, out_ref):
    tb = pl.program_id(1)
    b = pl.program_id(0)

    @pl.when(tb == 0)
    def _():
        out_ref[...] = jnp.zeros_like(out_ref)

    base = pl.multiple_of(base_ref[b, tb], 8)
    local = vseg_ref[0][0] - base
    srange = jax.lax.broadcasted_iota(jnp.int32, (_SBLK, _T), 0)
    onehot = (srange == local[None, :]).astype(jnp.float32)
    partial = jnp.dot(onehot, h_ref[0], preferred_element_type=jnp.float32)
    out_ref[pl.ds(base, _SBLK), :] += partial


def _sc_body(meta_hbm, pack_hbm, hid_hbm, zer_hbm, out_hbm,
             acc, inbuf, zbuf, sidx_v, meta_v, sw0, sw1):
    c = lax.axis_index("c")
    s = lax.axis_index("s")

    qiota = lax.iota(jnp.int32, 16)

    def _extract(vec, j):
        return jnp.max(jnp.where(qiota == j, vec, jnp.int32(-2**31)))

    pltpu.sync_copy(zer_hbm, zbuf)

    def _window(g, carry):
        lb = c
        wl = g
        w = s + wl * 16
        p = g & 1

        @pl.when(wl == 0)
        def _():
            mo = pl.multiple_of((lb * 16 + s) * 128, 128)
            pltpu.sync_copy(meta_hbm.at[pl.ds(mo, 128)], meta_v)

        mrow = meta_v[pl.ds(wl * 16, 16)]
        t0 = _extract(mrow, 0)
        nsc = _extract(mrow, 1)

        o0 = pl.multiple_of((lb * _SEQ + w * _WSEG) * _D, _AW)

        @pl.when((g >= 2) & (p == 0))
        def _():
            pltpu.make_async_copy(
                zer_hbm, acc.at[pl.ds(0, _AW)], sw0).wait()

        @pl.when((g >= 2) & (p == 1))
        def _():
            pltpu.make_async_copy(
                zer_hbm, acc.at[pl.ds(_AW, _AW)], sw1).wait()

        @pl.when((nsc == 0) & (p == 0))
        def _():
            pltpu.make_async_copy(
                zbuf, out_hbm.at[pl.ds(o0, _AW)], sw0).start()

        @pl.when((nsc == 0) & (p == 1))
        def _():
            pltpu.make_async_copy(
                zbuf, out_hbm.at[pl.ds(o0, _AW)], sw1).start()

        @pl.when(nsc > 0)
        def _():
            pltpu.sync_copy(zer_hbm, acc.at[pl.ds(p * _AW, _AW)])

            def _batch(k, carry2):
                cb = pl.multiple_of(
                    ((_BTC + lb) * _SEQ + t0 + k * _SUP) * _D, _SUP * _D)
                pltpu.sync_copy(hid_hbm.at[pl.ds(cb, _SUP * _D)], inbuf)
                cp = pl.multiple_of(lb * _SEQ + t0 + k * _SUP, _SUP)
                pltpu.sync_copy(pack_hbm.at[pl.ds(cp, _SUP)], sidx_v)
                for h in range(2):
                    sv = sidx_v[pl.ds(h * 16, 16)]
                    for j in range(16):
                        sb = _extract(sv, j)

                        @pl.when((sb >> 16) == w)
                        def _():
                            base = jnp.full(
                                (16,),
                                p * _AW + (sb & 0xFFFF) * _D,
                                jnp.int32) + qiota

                            def _q2(q, carry3):
                                for u in range(32):
                                    off = q * 512 + u * 16
                                    plsc.addupdate_scatter(
                                        acc, [base + off],
                                        inbuf[pl.ds(
                                            (h * 16 + j) * _D + off, 16)])
                                return carry3
                            lax.fori_loop(0, 2, _q2, 0)
                return carry2
            lax.fori_loop(0, nsc, _batch, 0)

            @pl.when(p == 0)
            def _():
                pltpu.make_async_copy(
                    acc.at[pl.ds(0, _AW)],
                    out_hbm.at[pl.ds(o0, _AW)], sw0).start()

            @pl.when(p == 1)
            def _():
                pltpu.make_async_copy(
                    acc.at[pl.ds(_AW, _AW)],
                    out_hbm.at[pl.ds(o0, _AW)], sw1).start()
        return carry
    lax.fori_loop(0, _NWT, _window, 0)

    pltpu.make_async_copy(zer_hbm, acc.at[pl.ds(0, _AW)], sw0).wait()
    pltpu.make_async_copy(zer_hbm, acc.at[pl.ds(_AW, _AW)], sw1).wait()


def kernel(domains, hidden_layers, binary_preditions):
    del domains
    labels = binary_preditions
    is_B = (labels == 1).astype(jnp.int32)
    is_I = labels == 2
    cs = jnp.cumsum(is_B, axis=1)
    seg = cs - 1
    valid = ((is_B == 1) | is_I) & (seg >= 0)
    vseg = jnp.where(valid, seg, -1).astype(jnp.int32)

    seg_sc = seg[_BTC:]
    pack = jnp.where(valid[_BTC:],
                     (seg_sc // _WSEG) * 65536 + seg_sc % _WSEG, -1)
    pack = pack.astype(jnp.int32).reshape(-1)

    bounds = jnp.arange(_NW + 1, dtype=jnp.int32) * _WSEG
    below = (seg_sc[:, None, :] < bounds[None, :, None]).sum(-1)
    t_lo = (below[:, :-1] // _SUP) * _SUP
    nsc = (below[:, 1:] - t_lo + _SUP - 1) // _SUP
    fields = jnp.stack(
        [t_lo, nsc] + [jnp.zeros_like(t_lo)] * 14, axis=-1)
    meta = (fields.reshape(_BSC, _NWT, 16, 16)
            .transpose(0, 2, 1, 3)
            .reshape(-1).astype(jnp.int32))

    hid_sc = hidden_layers.reshape(-1)
    zer = jnp.zeros((_AW,), jnp.float32)

    mesh = plsc.VectorSubcoreMesh(core_axis_name="c", subcore_axis_name="s")
    sc = functools.partial(
        pl.kernel,
        mesh=mesh,
        compiler_params=pltpu.CompilerParams(needs_layout_passes=False),
        out_type=jax.ShapeDtypeStruct((_BSC * _SEQ * _D,), jnp.float32),
        scratch_types=[
            pltpu.VMEM((2 * _AW,), jnp.float32),
            pltpu.VMEM((_SUP * _D,), jnp.float32),
            pltpu.VMEM((_AW,), jnp.float32),
            pltpu.VMEM((_SUP,), jnp.int32),
            pltpu.VMEM((128,), jnp.int32),
            pltpu.SemaphoreType.DMA,
            pltpu.SemaphoreType.DMA,
        ],
    )(_sc_body)
    out_sc = sc(meta, pack, hid_sc, zer).reshape(_BSC * _SEQ, _D)

    cs_excl = jnp.concatenate(
        [jnp.zeros((_BSZ, 1), jnp.int32), cs[:, :-1]], axis=1)
    base_raw = cs_excl[:_BTC, :: _T] - 1
    base = jnp.minimum((jnp.maximum(base_raw, 0) // 8) * 8, _SEQ - _SBLK)
    base = base.astype(jnp.int32)
    vseg3 = vseg[:_BTC].reshape(_BTC * _NTB, 1, _T)

    out_tc = pl.pallas_call(
        _tc_kernel,
        grid=(_BTC, _NTB),
        in_specs=[
            pl.BlockSpec(memory_space=pltpu.SMEM),
            pl.BlockSpec((1, 1, _T), lambda b, tb: (b * _NTB + tb, 0, 0)),
            pl.BlockSpec((1, _T, _D), lambda b, tb: (b, tb, 0)),
        ],
        out_specs=pl.BlockSpec((_SEQ, _D), lambda b, tb: (b, 0)),
        out_shape=jax.ShapeDtypeStruct((_BTC * _SEQ, _D), jnp.float32),
    )(base, vseg3, hidden_layers)

    return jnp.concatenate([out_tc, out_sc], axis=0)


# speedup vs baseline: 2.4761x; 1.0642x over previous
import functools

import jax
import jax.numpy as jnp
from jax import lax
from jax.experimental import pallas as pl
from jax.experimental.pallas import tpu as pltpu
from jax.experimental.pallas import tpu_sc as plsc

_BSZ, _SEQ, _D = 8, 2048, 1024
_BTC = 7
_BSC = _BSZ - _BTC

_T = 256
_SBLK = _T + 16
_NTB = _SEQ // _T

_WSEG = 16
_NW = _SEQ // _WSEG
_NWT = _NW // 16
_AW = _WSEG * _D
_SUP = 32


def _tc_kernel(base_ref, vseg_ref, h_ref, out_ref):
    tb = pl.program_id(1)
    b = pl.program_id(0)

    @pl.when(tb == 0)
    def _():
        out_ref[...] = jnp.zeros_like(out_ref)

    base = pl.multiple_of(base_ref[b, tb], 8)
    local = vseg_ref[0][0] - base
    srange = jax.lax.broadcasted_iota(jnp.int32, (_SBLK, _T), 0)
    onehot = (srange == local[None, :]).astype(jnp.float32)
    partial = jnp.dot(onehot, h_ref[0], preferred_element_type=jnp.float32)
    out_ref[pl.ds(base, _SBLK), :] += partial


def _sc_body(meta_hbm, pack_hbm, hid_hbm, zer_hbm, out_hbm,
             acc, inbuf, zbuf, sidx_v, meta_v, sw0, sw1):
    c = lax.axis_index("c")
    s = lax.axis_index("s")

    qiota = lax.iota(jnp.int32, 16)

    def _extract(vec, j):
        return jnp.max(jnp.where(qiota == j, vec, jnp.int32(-2**31)))

    pltpu.sync_copy(zer_hbm, zbuf)

    def _window(g, carry):
        lb = 0
        wl = c * (_NWT // 2) + g
        w = s + wl * 16
        p = g & 1

        @pl.when(g == 0)
        def _():
            mo = pl.multiple_of((lb * 16 + s) * 128, 128)
            pltpu.sync_copy(meta_hbm.at[pl.ds(mo, 128)], meta_v)

        mrow = meta_v[pl.ds(wl * 16, 16)]
        t0 = _extract(mrow, 0)
        nsc = _extract(mrow, 1)

        o0 = pl.multiple_of((lb * _SEQ + w * _WSEG) * _D, _AW)

        @pl.when((g >= 2) & (p == 0))
        def _():
            pltpu.make_async_copy(
                zer_hbm, acc.at[pl.ds(0, _AW)], sw0).wait()

        @pl.when((g >= 2) & (p == 1))
        def _():
            pltpu.make_async_copy(
                zer_hbm, acc.at[pl.ds(_AW, _AW)], sw1).wait()

        @pl.when((nsc == 0) & (p == 0))
        def _():
            pltpu.make_async_copy(
                zbuf, out_hbm.at[pl.ds(o0, _AW)], sw0).start()

        @pl.when((nsc == 0) & (p == 1))
        def _():
            pltpu.make_async_copy(
                zbuf, out_hbm.at[pl.ds(o0, _AW)], sw1).start()

        @pl.when(nsc > 0)
        def _():
            pltpu.sync_copy(zer_hbm, acc.at[pl.ds(p * _AW, _AW)])

            def _batch(k, carry2):
                cb = pl.multiple_of(
                    ((_BTC + lb) * _SEQ + t0 + k * _SUP) * _D, _SUP * _D)
                pltpu.sync_copy(hid_hbm.at[pl.ds(cb, _SUP * _D)], inbuf)
                cp = pl.multiple_of(lb * _SEQ + t0 + k * _SUP, _SUP)
                pltpu.sync_copy(pack_hbm.at[pl.ds(cp, _SUP)], sidx_v)
                for h in range(2):
                    sv = sidx_v[pl.ds(h * 16, 16)]
                    for j in range(16):
                        sb = _extract(sv, j)

                        @pl.when((sb >> 16) == w)
                        def _():
                            base = jnp.full(
                                (16,),
                                p * _AW + (sb & 0xFFFF) * _D,
                                jnp.int32) + qiota

                            def _q2(q, carry3):
                                for u in range(32):
                                    off = q * 512 + u * 16
                                    plsc.addupdate_scatter(
                                        acc, [base + off],
                                        inbuf[pl.ds(
                                            (h * 16 + j) * _D + off, 16)])
                                return carry3
                            lax.fori_loop(0, 2, _q2, 0)
                return carry2
            lax.fori_loop(0, nsc, _batch, 0)

            @pl.when(p == 0)
            def _():
                pltpu.make_async_copy(
                    acc.at[pl.ds(0, _AW)],
                    out_hbm.at[pl.ds(o0, _AW)], sw0).start()

            @pl.when(p == 1)
            def _():
                pltpu.make_async_copy(
                    acc.at[pl.ds(_AW, _AW)],
                    out_hbm.at[pl.ds(o0, _AW)], sw1).start()
        return carry
    lax.fori_loop(0, _NWT // 2, _window, 0)

    pltpu.make_async_copy(zer_hbm, acc.at[pl.ds(0, _AW)], sw0).wait()
    pltpu.make_async_copy(zer_hbm, acc.at[pl.ds(_AW, _AW)], sw1).wait()


def kernel(domains, hidden_layers, binary_preditions):
    del domains
    labels = binary_preditions
    is_B = (labels == 1).astype(jnp.int32)
    is_I = labels == 2
    cs = jnp.cumsum(is_B, axis=1)
    seg = cs - 1
    valid = ((is_B == 1) | is_I) & (seg >= 0)
    vseg = jnp.where(valid, seg, -1).astype(jnp.int32)

    seg_sc = seg[_BTC:]
    pack = jnp.where(valid[_BTC:],
                     (seg_sc // _WSEG) * 65536 + seg_sc % _WSEG, -1)
    pack = pack.astype(jnp.int32).reshape(-1)

    bounds = jnp.arange(_NW + 1, dtype=jnp.int32) * _WSEG
    below = (seg_sc[:, None, :] < bounds[None, :, None]).sum(-1)
    t_lo = (below[:, :-1] // _SUP) * _SUP
    nsc = (below[:, 1:] - t_lo + _SUP - 1) // _SUP
    fields = jnp.stack(
        [t_lo, nsc] + [jnp.zeros_like(t_lo)] * 14, axis=-1)
    meta = (fields.reshape(_BSC, _NWT, 16, 16)
            .transpose(0, 2, 1, 3)
            .reshape(-1).astype(jnp.int32))

    hid_sc = hidden_layers.reshape(-1)
    zer = jnp.zeros((_AW,), jnp.float32)

    mesh = plsc.VectorSubcoreMesh(core_axis_name="c", subcore_axis_name="s")
    sc = functools.partial(
        pl.kernel,
        mesh=mesh,
        compiler_params=pltpu.CompilerParams(needs_layout_passes=False),
        out_type=jax.ShapeDtypeStruct((_BSC * _SEQ * _D,), jnp.float32),
        scratch_types=[
            pltpu.VMEM((2 * _AW,), jnp.float32),
            pltpu.VMEM((_SUP * _D,), jnp.float32),
            pltpu.VMEM((_AW,), jnp.float32),
            pltpu.VMEM((_SUP,), jnp.int32),
            pltpu.VMEM((128,), jnp.int32),
            pltpu.SemaphoreType.DMA,
            pltpu.SemaphoreType.DMA,
        ],
    )(_sc_body)
    out_sc = sc(meta, pack, hid_sc, zer).reshape(_BSC * _SEQ, _D)

    cs_excl = jnp.concatenate(
        [jnp.zeros((_BSZ, 1), jnp.int32), cs[:, :-1]], axis=1)
    base_raw = cs_excl[:_BTC, :: _T] - 1
    base = jnp.minimum((jnp.maximum(base_raw, 0) // 8) * 8, _SEQ - _SBLK)
    base = base.astype(jnp.int32)
    vseg3 = vseg[:_BTC].reshape(_BTC * _NTB, 1, _T)

    out_tc = pl.pallas_call(
        _tc_kernel,
        grid=(_BTC, _NTB),
        in_specs=[
            pl.BlockSpec(memory_space=pltpu.SMEM),
            pl.BlockSpec((1, 1, _T), lambda b, tb: (b * _NTB + tb, 0, 0)),
            pl.BlockSpec((1, _T, _D), lambda b, tb: (b, tb, 0)),
        ],
        out_specs=pl.BlockSpec((_SEQ, _D), lambda b, tb: (b, 0)),
        out_shape=jax.ShapeDtypeStruct((_BTC * _SEQ, _D), jnp.float32),
    )(base, vseg3, hidden_layers)

    return jnp.concatenate([out_tc, out_sc], axis=0)
